# baseline (speedup 1.0000x reference)
import jax
import jax.numpy as jnp
from jax import lax
from jax.experimental import pallas as pl
from jax.experimental.pallas import tpu as pltpu
from jax.experimental.pallas import tpu_sc as plsc

_B = 16384
_D = 200
_V = 100000
_R = 500
_NC = 2
_NS = 16
_NW = _NC * _NS
_BPW = _B // _NW
_K = 4
_M = _BPW // _K


def _gather_body(ent_hbm, rel_hbm, e1_idx_hbm, rel_idx_hbm, out_hbm,
                 idx_v, rows_v, sem):
    wid = lax.axis_index("s") * _NC + lax.axis_index("c")
    base = wid * _BPW

    pltpu.sync_copy(e1_idx_hbm.at[wid], idx_v)
    cps = [pltpu.async_copy(ent_hbm.at[idx_v.at[j]],
                            rows_v.at[pl.ds(j * _M, _M)], sem)
           for j in range(_K)]
    for c in cps:
        c.wait()
    pltpu.sync_copy(rows_v, out_hbm.at[pl.ds(base, _BPW), pl.ds(0, _D)])

    pltpu.sync_copy(rel_idx_hbm.at[wid], idx_v)
    cps = [pltpu.async_copy(rel_hbm.at[idx_v.at[j]],
                            rows_v.at[pl.ds(j * _M, _M)], sem)
           for j in range(_K)]
    for c in cps:
        c.wait()
    pltpu.sync_copy(rows_v, out_hbm.at[pl.ds(base, _BPW), pl.ds(_D, _D)])


def _gather(ent3, rel3, e1_idx, rel_idx):
    mesh = plsc.VectorSubcoreMesh(core_axis_name="c", subcore_axis_name="s")
    f = pl.kernel(
        _gather_body,
        mesh=mesh,
        out_type=jax.ShapeDtypeStruct((_B, 2 * _D), jnp.float32),
        scratch_types=[
            pltpu.VMEM((_K, _M), jnp.int32),
            pltpu.VMEM((_BPW, _D), jnp.float32),
            pltpu.SemaphoreType.DMA,
        ],
        compiler_params=pltpu.CompilerParams(use_tc_tiling_on_sc=False),
    )
    return f(ent3, rel3,
             e1_idx.reshape(_NW, _K, _M), rel_idx.reshape(_NW, _K, _M))


def kernel(ent_table, rel_table, e1_idx, rel_idx):
    ent_flat = lax.optimization_barrier(ent_table.reshape(-1))
    rel_flat = lax.optimization_barrier(rel_table.reshape(-1))
    ent2 = ent_flat.reshape(_V, _D)
    rel2 = rel_flat.reshape(_R, _D)
    out = _gather(ent2, rel2, e1_idx, rel_idx)
    return out.reshape(_B, 1, 20, 20)


# speedup vs baseline: 1.8004x; 1.8004x over previous
import jax
import jax.numpy as jnp
from jax import lax
from jax.experimental import pallas as pl
from jax.experimental.pallas import tpu as pltpu
from jax.experimental.pallas import tpu_sc as plsc

_B = 16384
_D = 200
_V = 100000
_NC = 2
_NS = 16
_NW = _NC * _NS
_BPW = _B // _NW
_K = 4
_M = _BPW // _K


def _gather_half(table_hbm, idx_v, rows_v, sem):
    cps = [pltpu.async_copy(table_hbm.at[idx_v.at[j]],
                            rows_v.at[pl.ds(j * _M, _M)], sem)
           for j in range(_K)]
    for c in cps:
        c.wait()


def _gather_body(entA, entB, relA, relB, e1_idx_hbm, rel_idx_hbm, out_hbm,
                 idx_v, rows_v, sem):
    wid = lax.axis_index("s") * _NC + lax.axis_index("c")
    base = wid * _BPW

    pltpu.sync_copy(e1_idx_hbm.at[pl.ds(4 * wid, 4)], idx_v)
    _gather_half(entA, idx_v, rows_v, sem)
    pltpu.sync_copy(rows_v, out_hbm.at[pl.ds(base, _BPW), pl.ds(0, 128)])
    _gather_half(entB, idx_v, rows_v, sem)
    pltpu.sync_copy(rows_v.at[:, pl.ds(0, 72)],
                    out_hbm.at[pl.ds(base, _BPW), pl.ds(128, 72)])

    pltpu.sync_copy(rel_idx_hbm.at[pl.ds(4 * wid, 4)], idx_v)
    _gather_half(relA, idx_v, rows_v, sem)
    pltpu.sync_copy(rows_v, out_hbm.at[pl.ds(base, _BPW), pl.ds(200, 128)])
    _gather_half(relB, idx_v, rows_v, sem)
    pltpu.sync_copy(rows_v.at[:, pl.ds(0, 72)],
                    out_hbm.at[pl.ds(base, _BPW), pl.ds(328, 72)])


def _gather(entA, entB, relA, relB, e1m, rlm):
    mesh = plsc.VectorSubcoreMesh(core_axis_name="c", subcore_axis_name="s")
    f = pl.kernel(
        _gather_body,
        mesh=mesh,
        out_type=jax.ShapeDtypeStruct((_B, 2 * _D), jnp.float32),
        scratch_types=[
            pltpu.VMEM((_K, _M), jnp.int32),
            pltpu.VMEM((_BPW, 128), jnp.float32),
            pltpu.SemaphoreType.DMA,
        ],
        compiler_params=pltpu.CompilerParams(use_tc_tiling_on_sc=False),
    )
    return f(entA, entB, relA, relB, e1m, rlm)


def kernel(ent_table, rel_table, e1_idx, rel_idx):
    entA = ent_table[:, :128]
    entB = jnp.pad(ent_table[:, 128:], ((0, 0), (0, 56)))
    relp = jnp.pad(rel_table, ((0, 12), (0, 0)))
    relA = relp[:, :128]
    relB = jnp.pad(relp[:, 128:], ((0, 0), (0, 56)))
    e1m = e1_idx.reshape(128, 128)
    rlm = rel_idx.reshape(128, 128)
    out = _gather(entA, entB, relA, relB, e1m, rlm)
    return out.reshape(_B, 1, 20, 20)


# speedup vs baseline: 1.9482x; 1.0821x over previous
import jax
import jax.numpy as jnp
from jax import lax
from jax.experimental import pallas as pl
from jax.experimental.pallas import tpu as pltpu
from jax.experimental.pallas import tpu_sc as plsc

_B = 16384
_D = 200
_NC = 2
_NS = 16
_NW = _NC * _NS
_BPW = _B // _NW
_K = 4
_M = _BPW // _K


def _gather_half(table_hbm, idx_v, rows_v, sem):
    cps = [pltpu.async_copy(table_hbm.at[idx_v.at[pl.ds(j * _M, _M)]],
                            rows_v.at[pl.ds(j * _M, _M)], sem)
           for j in range(_K)]
    for c in cps:
        c.wait()


def _gather_body(entA, entB, relA, relB, e1_idx_hbm, rel_idx_hbm, out_hbm,
                 idx_v, rows_v, sem):
    wid = lax.axis_index("s") * _NC + lax.axis_index("c")
    base = wid * _BPW

    pltpu.sync_copy(e1_idx_hbm.at[pl.ds(base, _BPW)], idx_v)
    for col, half in ((0, entA), (128, entB)):
        _gather_half(half, idx_v, rows_v, sem)
        pltpu.sync_copy(rows_v, out_hbm.at[pl.ds(base, _BPW), pl.ds(col, 128)])

    pltpu.sync_copy(rel_idx_hbm.at[pl.ds(base, _BPW)], idx_v)
    for col, half in ((256, relA), (384, relB)):
        _gather_half(half, idx_v, rows_v, sem)
        pltpu.sync_copy(rows_v, out_hbm.at[pl.ds(base, _BPW), pl.ds(col, 128)])


def _gather(entA, entB, relA, relB, e1_idx, rel_idx):
    mesh = plsc.VectorSubcoreMesh(core_axis_name="c", subcore_axis_name="s")
    f = pl.kernel(
        _gather_body,
        mesh=mesh,
        out_type=jax.ShapeDtypeStruct((_B, 512), jnp.float32),
        scratch_types=[
            pltpu.VMEM((_BPW,), jnp.int32),
            pltpu.VMEM((_BPW, 128), jnp.float32),
            pltpu.SemaphoreType.DMA,
        ],
    )
    return f(entA, entB, relA, relB, e1_idx, rel_idx)


def kernel(ent_table, rel_table, e1_idx, rel_idx):
    entA = ent_table[:, :128]
    entB = jnp.pad(ent_table[:, 128:], ((0, 0), (0, 56)))
    relp = jnp.pad(rel_table, ((0, 12), (0, 0)))
    relA = relp[:, :128]
    relB = jnp.pad(relp[:, 128:], ((0, 0), (0, 56)))
    out512 = _gather(entA, entB, relA, relB, e1_idx, rel_idx)
    out = jnp.concatenate([out512[:, :200], out512[:, 256:456]], axis=1)
    return out.reshape(_B, 1, 20, 20)
